# initial kernel scaffold (unmeasured)
import jax
import jax.numpy as jnp
from jax import lax
from jax.experimental import pallas as pl
from jax.experimental.pallas import tpu as pltpu

N_DEV = 4
M_PER = 1024
N_OUT = 2048


def kernel(x, w_mat):
    m_full, k_per = x.shape
    _, n = w_mat.shape
    assert m_full == N_DEV * M_PER and n == N_OUT

    def body(x_ref, w_ref, out_ref, comm_ref, partial_ref, amax_ref,
             send_sems, recv_sems, amax_send_sems, amax_recv_sems,
             credit_sem, exit_sem):
        my = lax.axis_index("i")
        right = lax.rem(my + 1, N_DEV)
        left = lax.rem(my + N_DEV - 1, N_DEV)

        barrier_sem = pltpu.get_barrier_semaphore()
        for k in range(1, N_DEV):
            pl.semaphore_signal(
                barrier_sem, inc=1,
                device_id=(lax.rem(my + k, N_DEV),),
                device_id_type=pl.DeviceIdType.MESH,
            )
        pl.semaphore_wait(barrier_sem, N_DEV - 1)

        c0 = lax.rem(my + N_DEV - 1, N_DEV)
        comm_ref[0, :, :] = jnp.dot(
            x_ref[pl.ds(c0 * M_PER, M_PER), :], w_ref[:, :],
            preferred_element_type=jnp.float32,
        )

        for s in range(N_DEV - 1):
            slot = s % 2
            rslot = (s + 1) % 2
            if s >= 1:
                pl.semaphore_wait(credit_sem, 1)
            rdma = pltpu.make_async_remote_copy(
                src_ref=comm_ref.at[slot],
                dst_ref=comm_ref.at[rslot],
                send_sem=send_sems.at[slot],
                recv_sem=recv_sems.at[rslot],
                device_id=(right,),
                device_id_type=pl.DeviceIdType.MESH,
            )
            rdma.start()
            c = lax.rem(my + 2 * N_DEV - 2 - s, N_DEV)
            partial_ref[:, :] = jnp.dot(
                x_ref[pl.ds(c * M_PER, M_PER), :], w_ref[:, :],
                preferred_element_type=jnp.float32,
            )
            rdma.wait()
            pl.semaphore_signal(
                credit_sem, inc=1,
                device_id=(left,),
                device_id_type=pl.DeviceIdType.MESH,
            )
            if s < N_DEV - 2:
                comm_ref[rslot, :, :] = comm_ref[rslot, :, :] + partial_ref[:, :]
            else:
                out_ref[:, :] = comm_ref[rslot, :, :] + partial_ref[:, :]

        local_max = jnp.max(jnp.abs(out_ref[:, :]))
        amax_ref[pl.ds(my, 1), :, :] = jnp.full(
            (1, 8, 128), local_max, dtype=jnp.float32
        )
        sends = []
        for k in range(1, N_DEV):
            tgt = lax.rem(my + k, N_DEV)
            snd = pltpu.make_async_remote_copy(
                src_ref=amax_ref.at[my],
                dst_ref=amax_ref.at[my],
                send_sem=amax_send_sems.at[k - 1],
                recv_sem=amax_recv_sems.at[my],
                device_id=(tgt,),
                device_id_type=pl.DeviceIdType.MESH,
            )
            snd.start()
            sends.append(snd)
        for k in range(1, N_DEV):
            j = lax.rem(my + k, N_DEV)
            rcv = pltpu.make_async_remote_copy(
                src_ref=amax_ref.at[j],
                dst_ref=amax_ref.at[j],
                send_sem=amax_send_sems.at[k - 1],
                recv_sem=amax_recv_sems.at[j],
                device_id=(j,),
                device_id_type=pl.DeviceIdType.MESH,
            )
            rcv.wait_recv()
        for snd in sends:
            snd.wait_send()

        gmax = jnp.max(amax_ref[:, :, :])
        scale = gmax / 127.0
        q = jnp.clip(jnp.round(out_ref[:, :] / scale), -127.0, 127.0)
        out_ref[:, :] = q * scale

        for k in range(1, N_DEV):
            pl.semaphore_signal(
                exit_sem, inc=1,
                device_id=(lax.rem(my + k, N_DEV),),
                device_id_type=pl.DeviceIdType.MESH,
            )
        pl.semaphore_wait(exit_sem, N_DEV - 1)

    return pl.pallas_call(
        body,
        out_shape=jax.ShapeDtypeStruct((M_PER, N_OUT), jnp.float32),
        in_specs=[
            pl.BlockSpec(memory_space=pltpu.VMEM),
            pl.BlockSpec(memory_space=pltpu.VMEM),
        ],
        out_specs=pl.BlockSpec(memory_space=pltpu.VMEM),
        scratch_shapes=[
            pltpu.VMEM((2, M_PER, N_OUT), jnp.float32),
            pltpu.VMEM((M_PER, N_OUT), jnp.float32),
            pltpu.VMEM((N_DEV, 8, 128), jnp.float32),
            pltpu.SemaphoreType.DMA((2,)),
            pltpu.SemaphoreType.DMA((2,)),
            pltpu.SemaphoreType.DMA((3,)),
            pltpu.SemaphoreType.DMA((N_DEV,)),
            pltpu.SemaphoreType.REGULAR,
            pltpu.SemaphoreType.REGULAR,
        ],
        compiler_params=pltpu.CompilerParams(collective_id=0),
    )(x, w_mat)


# baseline (device time: 308356 ns/iter reference)
import jax
import jax.numpy as jnp
from jax import lax
from jax.experimental import pallas as pl
from jax.experimental.pallas import tpu as pltpu

N_DEV = 4
M_PER = 1024
N_OUT = 2048


def kernel(x, w_mat):
    def body(x_ref, w_ref, out_ref, comm_ref, partial_ref,
             my_amax_ref, amax_rx_ref,
             send_sems, recv_sems, amax_send_sems, amax_recv_sems):
        my = lax.axis_index("i")
        right = lax.rem(my + 1, N_DEV)
        left = lax.rem(my + N_DEV - 1, N_DEV)

        barrier_sem = pltpu.get_barrier_semaphore()
        for nbr in [left, right]:
            pl.semaphore_signal(
                barrier_sem, inc=1,
                device_id=(nbr,),
                device_id_type=pl.DeviceIdType.MESH,
            )
        pl.semaphore_wait(barrier_sem, 2)

        c0 = lax.rem(my + N_DEV - 1, N_DEV)
        comm_ref[0, :, :] = jnp.dot(
            x_ref[pl.ds(c0 * M_PER, M_PER), :], w_ref[:, :],
            preferred_element_type=jnp.float32,
        )

        for s in range(N_DEV - 1):
            slot = s % 2
            rslot = (s + 1) % 2
            rdma = pltpu.make_async_remote_copy(
                src_ref=comm_ref.at[slot],
                dst_ref=comm_ref.at[rslot],
                send_sem=send_sems.at[slot],
                recv_sem=recv_sems.at[rslot],
                device_id=(right,),
                device_id_type=pl.DeviceIdType.MESH,
            )
            rdma.start()
            c = lax.rem(my + 2 * N_DEV - 2 - s, N_DEV)
            partial_ref[:, :] = jnp.dot(
                x_ref[pl.ds(c * M_PER, M_PER), :], w_ref[:, :],
                preferred_element_type=jnp.float32,
            )
            rdma.wait()
            if s < N_DEV - 2:
                comm_ref[rslot, :, :] = comm_ref[rslot, :, :] + partial_ref[:, :]
            else:
                out_ref[:, :] = comm_ref[rslot, :, :] + partial_ref[:, :]

        local_max = jnp.max(jnp.abs(out_ref[:, :]))
        my_amax_ref[:, :] = jnp.full((8, 128), local_max, dtype=jnp.float32)
        sends = []
        for k in range(1, N_DEV):
            tgt = lax.rem(my + k, N_DEV)
            snd = pltpu.make_async_remote_copy(
                src_ref=my_amax_ref,
                dst_ref=amax_rx_ref.at[k - 1],
                send_sem=amax_send_sems.at[k - 1],
                recv_sem=amax_recv_sems.at[k - 1],
                device_id=(tgt,),
                device_id_type=pl.DeviceIdType.MESH,
            )
            snd.start()
            sends.append(snd)
        for k in range(1, N_DEV):
            rcv = pltpu.make_async_remote_copy(
                src_ref=my_amax_ref,
                dst_ref=amax_rx_ref.at[k - 1],
                send_sem=amax_send_sems.at[k - 1],
                recv_sem=amax_recv_sems.at[k - 1],
                device_id=(my,),
                device_id_type=pl.DeviceIdType.MESH,
            )
            rcv.wait_recv()
        for snd in sends:
            snd.wait_send()

        gmax = jnp.maximum(local_max, jnp.max(amax_rx_ref[:, :, :]))
        scale = gmax / 127.0
        q = jnp.clip(jnp.round(out_ref[:, :] / scale), -127.0, 127.0)
        out_ref[:, :] = q * scale

    return pl.pallas_call(
        body,
        out_shape=jax.ShapeDtypeStruct((M_PER, N_OUT), jnp.float32),
        in_specs=[
            pl.BlockSpec(memory_space=pltpu.VMEM),
            pl.BlockSpec(memory_space=pltpu.VMEM),
        ],
        out_specs=pl.BlockSpec(memory_space=pltpu.VMEM),
        scratch_shapes=[
            pltpu.VMEM((2, M_PER, N_OUT), jnp.float32),
            pltpu.VMEM((M_PER, N_OUT), jnp.float32),
            pltpu.VMEM((8, 128), jnp.float32),
            pltpu.VMEM((N_DEV - 1, 8, 128), jnp.float32),
            pltpu.SemaphoreType.DMA((2,)),
            pltpu.SemaphoreType.DMA((2,)),
            pltpu.SemaphoreType.DMA((3,)),
            pltpu.SemaphoreType.DMA((3,)),
        ],
        compiler_params=pltpu.CompilerParams(
            collective_id=0,
            vmem_limit_bytes=100 * 1024 * 1024,
        ),
    )(x, w_mat)


# device time: 107735 ns/iter; 2.8622x vs baseline; 2.8622x over previous
import jax
import jax.numpy as jnp
from jax import lax
from jax.experimental import pallas as pl
from jax.experimental.pallas import tpu as pltpu

N_DEV = 4
M_PER = 1024
N_OUT = 2048
N_HALF = N_OUT // 2


def kernel(x, w_mat):
    def body(x_ref, w_ref, out_ref, cw_ref, ccw_ref, pcw_ref, pccw_ref,
             my_amax_ref, amax_rx_ref,
             cw_send_sems, cw_recv_sems, ccw_send_sems, ccw_recv_sems,
             amax_send_sems, amax_recv_sems):
        my = lax.axis_index("i")
        right = lax.rem(my + 1, N_DEV)
        left = lax.rem(my + N_DEV - 1, N_DEV)

        barrier_sem = pltpu.get_barrier_semaphore()
        for nbr in [left, right]:
            pl.semaphore_signal(
                barrier_sem, inc=1,
                device_id=(nbr,),
                device_id_type=pl.DeviceIdType.MESH,
            )
        pl.semaphore_wait(barrier_sem, 2)

        c_cw = lax.rem(my + N_DEV - 1, N_DEV)
        c_ccw = lax.rem(my + 1, N_DEV)
        cw_ref[0, :, :] = jnp.dot(
            x_ref[pl.ds(c_cw * M_PER, M_PER), :], w_ref[:, :N_HALF],
            preferred_element_type=jnp.float32,
        ).astype(jnp.bfloat16)
        ccw_ref[0, :, :] = jnp.dot(
            x_ref[pl.ds(c_ccw * M_PER, M_PER), :], w_ref[:, N_HALF:],
            preferred_element_type=jnp.float32,
        ).astype(jnp.bfloat16)

        for s in range(N_DEV - 1):
            slot = s % 2
            rslot = (s + 1) % 2
            cw_rdma = pltpu.make_async_remote_copy(
                src_ref=cw_ref.at[slot],
                dst_ref=cw_ref.at[rslot],
                send_sem=cw_send_sems.at[slot],
                recv_sem=cw_recv_sems.at[rslot],
                device_id=(right,),
                device_id_type=pl.DeviceIdType.MESH,
            )
            cw_rdma.start()
            ccw_rdma = pltpu.make_async_remote_copy(
                src_ref=ccw_ref.at[slot],
                dst_ref=ccw_ref.at[rslot],
                send_sem=ccw_send_sems.at[slot],
                recv_sem=ccw_recv_sems.at[rslot],
                device_id=(left,),
                device_id_type=pl.DeviceIdType.MESH,
            )
            ccw_rdma.start()

            c_cw = lax.rem(my + 2 * N_DEV - 2 - s, N_DEV)
            c_ccw = lax.rem(my + 2 + s, N_DEV)
            pcw_ref[:, :] = jnp.dot(
                x_ref[pl.ds(c_cw * M_PER, M_PER), :], w_ref[:, :N_HALF],
                preferred_element_type=jnp.float32,
            )
            pccw_ref[:, :] = jnp.dot(
                x_ref[pl.ds(c_ccw * M_PER, M_PER), :], w_ref[:, N_HALF:],
                preferred_element_type=jnp.float32,
            )
            cw_rdma.wait()
            ccw_rdma.wait()
            if s < N_DEV - 2:
                cw_ref[rslot, :, :] = (
                    cw_ref[rslot, :, :].astype(jnp.float32) + pcw_ref[:, :]
                ).astype(jnp.bfloat16)
                ccw_ref[rslot, :, :] = (
                    ccw_ref[rslot, :, :].astype(jnp.float32) + pccw_ref[:, :]
                ).astype(jnp.bfloat16)
            else:
                out_ref[:, :N_HALF] = (
                    cw_ref[rslot, :, :].astype(jnp.float32) + pcw_ref[:, :]
                )
                out_ref[:, N_HALF:] = (
                    ccw_ref[rslot, :, :].astype(jnp.float32) + pccw_ref[:, :]
                )

        local_max = jnp.max(jnp.abs(out_ref[:, :]))
        my_amax_ref[:, :] = jnp.full((8, 128), local_max, dtype=jnp.float32)
        sends = []
        for k in range(1, N_DEV):
            tgt = lax.rem(my + k, N_DEV)
            snd = pltpu.make_async_remote_copy(
                src_ref=my_amax_ref,
                dst_ref=amax_rx_ref.at[k - 1],
                send_sem=amax_send_sems.at[k - 1],
                recv_sem=amax_recv_sems.at[k - 1],
                device_id=(tgt,),
                device_id_type=pl.DeviceIdType.MESH,
            )
            snd.start()
            sends.append(snd)
        for k in range(1, N_DEV):
            rcv = pltpu.make_async_remote_copy(
                src_ref=my_amax_ref,
                dst_ref=amax_rx_ref.at[k - 1],
                send_sem=amax_send_sems.at[k - 1],
                recv_sem=amax_recv_sems.at[k - 1],
                device_id=(my,),
                device_id_type=pl.DeviceIdType.MESH,
            )
            rcv.wait_recv()
        for snd in sends:
            snd.wait_send()

        gmax = jnp.maximum(local_max, jnp.max(amax_rx_ref[:, :, :]))
        scale = gmax / 127.0
        q = jnp.clip(jnp.round(out_ref[:, :] / scale), -127.0, 127.0)
        out_ref[:, :] = q * scale

    return pl.pallas_call(
        body,
        out_shape=jax.ShapeDtypeStruct((M_PER, N_OUT), jnp.float32),
        in_specs=[
            pl.BlockSpec(memory_space=pltpu.VMEM),
            pl.BlockSpec(memory_space=pltpu.VMEM),
        ],
        out_specs=pl.BlockSpec(memory_space=pltpu.VMEM),
        scratch_shapes=[
            pltpu.VMEM((2, M_PER, N_HALF), jnp.bfloat16),
            pltpu.VMEM((2, M_PER, N_HALF), jnp.bfloat16),
            pltpu.VMEM((M_PER, N_HALF), jnp.float32),
            pltpu.VMEM((M_PER, N_HALF), jnp.float32),
            pltpu.VMEM((8, 128), jnp.float32),
            pltpu.VMEM((N_DEV - 1, 8, 128), jnp.float32),
            pltpu.SemaphoreType.DMA((2,)),
            pltpu.SemaphoreType.DMA((2,)),
            pltpu.SemaphoreType.DMA((2,)),
            pltpu.SemaphoreType.DMA((2,)),
            pltpu.SemaphoreType.DMA((3,)),
            pltpu.SemaphoreType.DMA((3,)),
        ],
        compiler_params=pltpu.CompilerParams(
            collective_id=0,
            vmem_limit_bytes=100 * 1024 * 1024,
        ),
    )(x, w_mat)


# device time: 106631 ns/iter; 2.8918x vs baseline; 1.0104x over previous
import jax
import jax.numpy as jnp
from jax import lax
from jax.experimental import pallas as pl
from jax.experimental.pallas import tpu as pltpu

N_DEV = 4
M_PER = 1024
N_OUT = 2048
N_HALF = N_OUT // 2


def kernel(x, w_mat):
    def body(x_ref, w_ref, out_ref, cw_ref, ccw_ref, pcw_ref, pccw_ref,
             my_amax_ref, amax_rx_ref,
             cw_send_sems, cw_recv_sems, ccw_send_sems, ccw_recv_sems,
             amax_send_sems, amax_recv_sems):
        my = lax.axis_index("i")
        right = lax.rem(my + 1, N_DEV)
        left = lax.rem(my + N_DEV - 1, N_DEV)

        barrier_sem = pltpu.get_barrier_semaphore()
        for nbr in [left, right]:
            pl.semaphore_signal(
                barrier_sem, inc=1,
                device_id=(nbr,),
                device_id_type=pl.DeviceIdType.MESH,
            )
        pl.semaphore_wait(barrier_sem, 2)

        c_cw = lax.rem(my + N_DEV - 1, N_DEV)
        c_ccw = lax.rem(my + 1, N_DEV)
        cw_ref[0, :, :] = jnp.dot(
            x_ref[pl.ds(c_cw * M_PER, M_PER), :], w_ref[:, :N_HALF],
            preferred_element_type=jnp.float32,
        ).astype(jnp.bfloat16)
        ccw_ref[0, :, :] = jnp.dot(
            x_ref[pl.ds(c_ccw * M_PER, M_PER), :], w_ref[:, N_HALF:],
            preferred_element_type=jnp.float32,
        ).astype(jnp.bfloat16)

        for s in range(N_DEV - 1):
            slot = s % 2
            rslot = (s + 1) % 2
            cw_rdma = pltpu.make_async_remote_copy(
                src_ref=cw_ref.at[slot],
                dst_ref=cw_ref.at[rslot],
                send_sem=cw_send_sems.at[slot],
                recv_sem=cw_recv_sems.at[rslot],
                device_id=(right,),
                device_id_type=pl.DeviceIdType.MESH,
            )
            cw_rdma.start()
            ccw_rdma = pltpu.make_async_remote_copy(
                src_ref=ccw_ref.at[slot],
                dst_ref=ccw_ref.at[rslot],
                send_sem=ccw_send_sems.at[slot],
                recv_sem=ccw_recv_sems.at[rslot],
                device_id=(left,),
                device_id_type=pl.DeviceIdType.MESH,
            )
            ccw_rdma.start()

            c_cw = lax.rem(my + 2 * N_DEV - 2 - s, N_DEV)
            c_ccw = lax.rem(my + 2 + s, N_DEV)
            pcw_ref[:, :] = jnp.dot(
                x_ref[pl.ds(c_cw * M_PER, M_PER), :], w_ref[:, :N_HALF],
                preferred_element_type=jnp.float32,
            )
            pccw_ref[:, :] = jnp.dot(
                x_ref[pl.ds(c_ccw * M_PER, M_PER), :], w_ref[:, N_HALF:],
                preferred_element_type=jnp.float32,
            )
            cw_rdma.wait()
            ccw_rdma.wait()
            if s < N_DEV - 2:
                cw_ref[rslot, :, :] = (
                    cw_ref[rslot, :, :].astype(jnp.float32) + pcw_ref[:, :]
                ).astype(jnp.bfloat16)
                ccw_ref[rslot, :, :] = (
                    ccw_ref[rslot, :, :].astype(jnp.float32) + pccw_ref[:, :]
                ).astype(jnp.bfloat16)
            else:
                out_ref[:, :N_HALF] = (
                    cw_ref[rslot, :, :].astype(jnp.float32) + pcw_ref[:, :]
                )
                out_ref[:, N_HALF:] = (
                    ccw_ref[rslot, :, :].astype(jnp.float32) + pccw_ref[:, :]
                )

        local_max = jnp.max(jnp.abs(out_ref[:, :]))
        my_amax_ref[:, :] = jnp.full((8, 128), local_max, dtype=jnp.float32)
        sends = []
        for k in range(1, N_DEV):
            tgt = lax.rem(my + k, N_DEV)
            snd = pltpu.make_async_remote_copy(
                src_ref=my_amax_ref,
                dst_ref=amax_rx_ref.at[k - 1],
                send_sem=amax_send_sems.at[k - 1],
                recv_sem=amax_recv_sems.at[k - 1],
                device_id=(tgt,),
                device_id_type=pl.DeviceIdType.MESH,
            )
            snd.start()
            sends.append(snd)
        for k in range(1, N_DEV):
            rcv = pltpu.make_async_remote_copy(
                src_ref=my_amax_ref,
                dst_ref=amax_rx_ref.at[k - 1],
                send_sem=amax_send_sems.at[k - 1],
                recv_sem=amax_recv_sems.at[k - 1],
                device_id=(my,),
                device_id_type=pl.DeviceIdType.MESH,
            )
            rcv.wait_recv()
        for snd in sends:
            snd.wait_send()

        gmax = jnp.maximum(local_max, jnp.max(amax_rx_ref[:, :, :]))
        scale = gmax / 127.0
        inv_scale = 1.0 / scale
        q = jnp.clip(jnp.round(out_ref[:, :] * inv_scale), -127.0, 127.0)
        out_ref[:, :] = q * scale

    return pl.pallas_call(
        body,
        out_shape=jax.ShapeDtypeStruct((M_PER, N_OUT), jnp.float32),
        in_specs=[
            pl.BlockSpec(memory_space=pltpu.VMEM),
            pl.BlockSpec(memory_space=pltpu.VMEM),
        ],
        out_specs=pl.BlockSpec(memory_space=pltpu.VMEM),
        scratch_shapes=[
            pltpu.VMEM((2, M_PER, N_HALF), jnp.bfloat16),
            pltpu.VMEM((2, M_PER, N_HALF), jnp.bfloat16),
            pltpu.VMEM((M_PER, N_HALF), jnp.float32),
            pltpu.VMEM((M_PER, N_HALF), jnp.float32),
            pltpu.VMEM((8, 128), jnp.float32),
            pltpu.VMEM((N_DEV - 1, 8, 128), jnp.float32),
            pltpu.SemaphoreType.DMA((2,)),
            pltpu.SemaphoreType.DMA((2,)),
            pltpu.SemaphoreType.DMA((2,)),
            pltpu.SemaphoreType.DMA((2,)),
            pltpu.SemaphoreType.DMA((3,)),
            pltpu.SemaphoreType.DMA((3,)),
        ],
        compiler_params=pltpu.CompilerParams(
            collective_id=0,
            vmem_limit_bytes=100 * 1024 * 1024,
        ),
    )(x, w_mat)


# device time: 98478 ns/iter; 3.1312x vs baseline; 1.0828x over previous
import jax
import jax.numpy as jnp
from jax import lax
from jax.experimental import pallas as pl
from jax.experimental.pallas import tpu as pltpu

N_DEV = 4
M_PER = 1024
N_OUT = 2048
H = 512


def kernel(x, w_mat):
    def body(x_ref, w_ref, out_ref,
             cwa_ref, ccwa_ref, cwb_ref, ccwb_ref,
             pcwa_ref, pccwa_ref, pcwb_ref, pccwb_ref,
             my_amax_ref, amax_rx_ref,
             cwa_ss, cwa_rs, ccwa_ss, ccwa_rs,
             cwb_ss, cwb_rs, ccwb_ss, ccwb_rs,
             amax_send_sems, amax_recv_sems):
        my = lax.axis_index("i")
        right = lax.rem(my + 1, N_DEV)
        left = lax.rem(my + N_DEV - 1, N_DEV)

        barrier_sem = pltpu.get_barrier_semaphore()
        for nbr in [left, right]:
            pl.semaphore_signal(
                barrier_sem, inc=1,
                device_id=(nbr,),
                device_id_type=pl.DeviceIdType.MESH,
            )
        pl.semaphore_wait(barrier_sem, 2)

        rings = [
            (cwa_ref, pcwa_ref, cwa_ss, cwa_rs, right, 0, True),
            (ccwa_ref, pccwa_ref, ccwa_ss, ccwa_rs, left, 2 * H, False),
            (cwb_ref, pcwb_ref, cwb_ss, cwb_rs, right, H, True),
            (ccwb_ref, pccwb_ref, ccwb_ss, ccwb_rs, left, 3 * H, False),
        ]

        def dot_block(c, col):
            return jnp.dot(
                x_ref[pl.ds(c * M_PER, M_PER), :], w_ref[:, col:col + H],
                preferred_element_type=jnp.float32,
            )

        def c_recv(s, is_cw):
            return lax.rem(my + (2 * N_DEV - 2 - s if is_cw else 2 + s),
                           N_DEV)

        c_cw0 = lax.rem(my + N_DEV - 1, N_DEV)
        c_ccw0 = lax.rem(my + 1, N_DEV)

        def make_rdma(ring, s):
            comm, _, ss, rs, dev, _, _ = ring
            return pltpu.make_async_remote_copy(
                src_ref=comm.at[s % 2],
                dst_ref=comm.at[(s + 1) % 2],
                send_sem=ss.at[s % 2],
                recv_sem=rs.at[(s + 1) % 2],
                device_id=(dev,),
                device_id_type=pl.DeviceIdType.MESH,
            )

        rdmas = [None] * 4
        for pair in ([0, 1], [2, 3]):
            for i in pair:
                comm, _, _, _, _, col, is_cw = rings[i]
                comm[0, :, :] = dot_block(
                    c_cw0 if is_cw else c_ccw0, col
                ).astype(jnp.bfloat16)
            for i in pair:
                rdmas[i] = make_rdma(rings[i], 0)
                rdmas[i].start()
        for i in range(4):
            _, part, _, _, _, col, is_cw = rings[i]
            part[:, :] = dot_block(c_recv(0, is_cw), col)

        for s in range(N_DEV - 1):
            rslot = (s + 1) % 2
            for i in range(4):
                comm, part, _, _, _, col, _ = rings[i]
                rdmas[i].wait()
                if s < N_DEV - 2:
                    comm[rslot, :, :] = (
                        comm[rslot, :, :].astype(jnp.float32) + part[:, :]
                    ).astype(jnp.bfloat16)
                    rdmas[i] = make_rdma(rings[i], s + 1)
                    rdmas[i].start()
                else:
                    out_ref[:, col:col + H] = (
                        comm[rslot, :, :].astype(jnp.float32) + part[:, :]
                    )
            if s < N_DEV - 2:
                for i in range(4):
                    _, part, _, _, _, col, is_cw = rings[i]
                    part[:, :] = dot_block(c_recv(s + 1, is_cw), col)

        local_max = jnp.max(jnp.abs(out_ref[:, :]))
        my_amax_ref[:, :] = jnp.full((8, 128), local_max, dtype=jnp.float32)
        sends = []
        for k in range(1, N_DEV):
            tgt = lax.rem(my + k, N_DEV)
            snd = pltpu.make_async_remote_copy(
                src_ref=my_amax_ref,
                dst_ref=amax_rx_ref.at[k - 1],
                send_sem=amax_send_sems.at[k - 1],
                recv_sem=amax_recv_sems.at[k - 1],
                device_id=(tgt,),
                device_id_type=pl.DeviceIdType.MESH,
            )
            snd.start()
            sends.append(snd)
        for k in range(1, N_DEV):
            rcv = pltpu.make_async_remote_copy(
                src_ref=my_amax_ref,
                dst_ref=amax_rx_ref.at[k - 1],
                send_sem=amax_send_sems.at[k - 1],
                recv_sem=amax_recv_sems.at[k - 1],
                device_id=(my,),
                device_id_type=pl.DeviceIdType.MESH,
            )
            rcv.wait_recv()
        for snd in sends:
            snd.wait_send()

        gmax = jnp.maximum(local_max, jnp.max(amax_rx_ref[:, :, :]))
        scale = gmax / 127.0
        inv_scale = 1.0 / scale
        q = jnp.clip(jnp.round(out_ref[:, :] * inv_scale), -127.0, 127.0)
        out_ref[:, :] = q * scale

    return pl.pallas_call(
        body,
        out_shape=jax.ShapeDtypeStruct((M_PER, N_OUT), jnp.float32),
        in_specs=[
            pl.BlockSpec(memory_space=pltpu.VMEM),
            pl.BlockSpec(memory_space=pltpu.VMEM),
        ],
        out_specs=pl.BlockSpec(memory_space=pltpu.VMEM),
        scratch_shapes=[
            pltpu.VMEM((2, M_PER, H), jnp.bfloat16),
            pltpu.VMEM((2, M_PER, H), jnp.bfloat16),
            pltpu.VMEM((2, M_PER, H), jnp.bfloat16),
            pltpu.VMEM((2, M_PER, H), jnp.bfloat16),
            pltpu.VMEM((M_PER, H), jnp.float32),
            pltpu.VMEM((M_PER, H), jnp.float32),
            pltpu.VMEM((M_PER, H), jnp.float32),
            pltpu.VMEM((M_PER, H), jnp.float32),
            pltpu.VMEM((8, 128), jnp.float32),
            pltpu.VMEM((N_DEV - 1, 8, 128), jnp.float32),
            pltpu.SemaphoreType.DMA((2,)),
            pltpu.SemaphoreType.DMA((2,)),
            pltpu.SemaphoreType.DMA((2,)),
            pltpu.SemaphoreType.DMA((2,)),
            pltpu.SemaphoreType.DMA((2,)),
            pltpu.SemaphoreType.DMA((2,)),
            pltpu.SemaphoreType.DMA((2,)),
            pltpu.SemaphoreType.DMA((2,)),
            pltpu.SemaphoreType.DMA((3,)),
            pltpu.SemaphoreType.DMA((3,)),
        ],
        compiler_params=pltpu.CompilerParams(
            collective_id=0,
            vmem_limit_bytes=100 * 1024 * 1024,
        ),
    )(x, w_mat)
